# Initial kernel scaffold; baseline (speedup 1.0000x reference)
import jax
import jax.numpy as jnp
from jax.experimental import pallas as pl


def kernel(idx, targets, table):
    raise NotImplementedError("write your pallas kernel here")


# speedup vs baseline: 2.2496x; 2.2496x over previous
import functools

import jax
import jax.numpy as jnp
from jax import lax
from jax.experimental import pallas as pl
from jax.experimental.pallas import tpu as pltpu
from jax.experimental.pallas import tpu_sc as plsc

V = 8192
N = 8192
NC, NS, L = 2, 16, 16
NW = NC * NS
BPW = N // NW
K = 4
NCHUNK = BPW // K
VREGS = V // L
UNROLL = 4


def _lanes_all_reduce(vec, op):
    lane = lax.broadcasted_iota(jnp.int32, (L,), 0)
    for k in (8, 4, 2, 1):
        vec = op(vec, vec.at[lane ^ k].get(mode="promise_in_bounds"))
    return vec


def _row_stats(buf, j):
    neg = jnp.full((L,), -jnp.inf, jnp.float32)
    zero = jnp.zeros((L,), jnp.float32)

    def pboth(k, accs):
        maxes, sums = accs
        base = k * (L * UNROLL)
        x = [buf[j, pl.ds(base + u * L, L)] for u in range(UNROLL)]
        return (
            tuple(jnp.maximum(maxes[u], x[u]) for u in range(UNROLL)),
            tuple(sums[u] + jnp.exp(x[u]) for u in range(UNROLL)),
        )

    maxes, sums = lax.fori_loop(
        0, VREGS // UNROLL, pboth, ((neg,) * UNROLL, (zero,) * UNROLL))
    m = _lanes_all_reduce(functools.reduce(jnp.maximum, maxes), jnp.maximum)
    s0 = _lanes_all_reduce(functools.reduce(lax.add, sums), lax.add)
    return m, s0


def _row_stats_shifted(buf, j, m):
    zero = jnp.zeros((L,), jnp.float32)

    def psum(k, accs):
        base = k * (L * UNROLL)
        return tuple(
            accs[u] + jnp.exp(buf[j, pl.ds(base + u * L, L)] - m)
            for u in range(UNROLL)
        )

    sums = lax.fori_loop(0, VREGS // UNROLL, psum, (zero,) * UNROLL)
    return _lanes_all_reduce(functools.reduce(lax.add, sums), lax.add)


def _sc_body(idxr_hbm, tgtf_hbm, table_hbm,
             out_hbm, m_hbm, s_hbm, p_hbm,
             idx2_v, tgt_v, m_v, s_v, p_v,
             buf0, buf1, sem_g0, sem_g1, sem_s0, sem_s1):
    wid = lax.axis_index("s") * NC + lax.axis_index("c")
    base = wid * BPW

    pltpu.sync_copy(idxr_hbm.at[wid], idx2_v)
    pltpu.sync_copy(tgtf_hbm.at[wid], tgt_v)

    bufs = (buf0, buf1)
    sems_g = (sem_g0, sem_g1)
    sems_s = (sem_s0, sem_s1)

    def gather(c, b):
        return pltpu.async_copy(table_hbm.at[idx2_v.at[c]], bufs[b], sems_g[b])

    def scatter_desc(c, b):
        return pltpu.make_async_copy(
            bufs[b], out_hbm.at[pl.ds(base + c * K, K)], sems_s[b])

    gather(0, 0)

    def outer(t4, acc_p):
        g16 = t4 * L
        tv = tgt_v[pl.ds(g16, L)]
        lane = lax.broadcasted_iota(jnp.int32, (L,), 0)

        for cc in range(4):
            c = 4 * t4 + cc
            b = cc % 2
            nxt = c + 1

            @pl.when(nxt < NCHUNK)
            def _():
                @pl.when(c >= 1)
                def _():
                    scatter_desc(c - 1, 1 - b).wait()

                gather(nxt, 1 - b)

            pltpu.make_async_copy(
                table_hbm.at[idx2_v.at[c]], bufs[b], sems_g[b]).wait()

            buf = bufs[b]

            for j in range(K):
                m, s0 = _row_stats(buf, j)
                rl = 4 * cc + j

                t = tv[rl]
                xv = buf[j, pl.ds((t // L) * L, L)]
                acc_p = acc_p + jnp.where(lane == t % L, xv, 0.0)

                msk = lane == rl
                m_v[pl.ds(g16, L)] = jnp.where(msk, 0.0, m_v[pl.ds(g16, L)])
                s_v[pl.ds(g16, L)] = jnp.where(msk, s0, s_v[pl.ds(g16, L)])

                @pl.when(jnp.abs(m[0]) > 60.0)
                def _():
                    s = _row_stats_shifted(buf, j, m)
                    m_v[pl.ds(g16, L)] = jnp.where(msk, m, m_v[pl.ds(g16, L)])
                    s_v[pl.ds(g16, L)] = jnp.where(msk, s, s_v[pl.ds(g16, L)])

            scatter_desc(c, b).start()
        return acc_p

    acc_p = lax.fori_loop(
        0, NCHUNK // 4, outer, jnp.zeros((L,), jnp.float32))
    scatter_desc(NCHUNK - 2, 0).wait()
    scatter_desc(NCHUNK - 1, 1).wait()

    p_v[...] = acc_p
    pltpu.sync_copy(m_v, m_hbm.at[pl.ds(base, BPW)])
    pltpu.sync_copy(s_v, s_hbm.at[pl.ds(base, BPW)])
    pltpu.sync_copy(p_v, p_hbm.at[pl.ds(wid * L, L)])


_sc_call = pl.kernel(
    _sc_body,
    out_type=(
        jax.ShapeDtypeStruct((N, V), jnp.float32),
        jax.ShapeDtypeStruct((N,), jnp.float32),
        jax.ShapeDtypeStruct((N,), jnp.float32),
        jax.ShapeDtypeStruct((NW * L,), jnp.float32),
    ),
    mesh=plsc.VectorSubcoreMesh(
        core_axis_name="c", subcore_axis_name="s",
        num_cores=NC, num_subcores=NS),
    scratch_types=(
        pltpu.VMEM((NCHUNK, K), jnp.int32),
        pltpu.VMEM((BPW,), jnp.int32),
        pltpu.VMEM((BPW,), jnp.float32),
        pltpu.VMEM((BPW,), jnp.float32),
        pltpu.VMEM((L,), jnp.float32),
        pltpu.VMEM((K, V), jnp.float32),
        pltpu.VMEM((K, V), jnp.float32),
        pltpu.SemaphoreType.DMA,
        pltpu.SemaphoreType.DMA,
        pltpu.SemaphoreType.DMA,
        pltpu.SemaphoreType.DMA,
    ),
)


def _loss_body(m_ref, s_ref, p_ref, o_ref):
    logz_sum = jnp.sum(m_ref[...] + jnp.log(s_ref[...]))
    o_ref[0, 0] = (logz_sum - jnp.sum(p_ref[...])) / N


_loss_call = pl.pallas_call(
    _loss_body,
    out_shape=jax.ShapeDtypeStruct((1, 1), jnp.float32),
    in_specs=[pl.BlockSpec(memory_space=pltpu.VMEM)] * 3,
    out_specs=pl.BlockSpec(memory_space=pltpu.SMEM),
)


def kernel(idx, targets, table):
    idx_r = idx.reshape(NW, NCHUNK, K)
    tgt_f = targets.reshape(NW, BPW)
    logits, m, s, picked = _sc_call(idx_r, tgt_f, table)
    loss = _loss_call(m.reshape(64, 128), s.reshape(64, 128),
                      picked.reshape(4, 128))
    return logits, loss.reshape(())
